# baseline (speedup 1.0000x reference)
import jax
import jax.numpy as jnp
from jax import lax
from jax.experimental import pallas as pl
from jax.experimental.pallas import tpu as pltpu
from jax.experimental.pallas import tpu_sc as plsc

NUM_CORES = 2
NUM_SUBCORES = 16
LANES = 16

B = 16384
F = 26
V = 100000
BH = B // NUM_CORES
ROWS = BH // 128
OROWS = ROWS // 8
NH = 4
HROWS = ROWS // NH
HB = BH // NH


def _lookup_body(xt_hbm, tab_hbm, bias_hbm, zer_hbm, out_hbm,
                 sub_v, idx_v, part_v, iota_v, bias_v, outb_v,
                 accum, sem_tab, sem_idx):
    c = lax.axis_index("c")
    s = lax.axis_index("s")

    for h in range(NH):
        iota_v[h, pl.ds(0, LANES)] = (
            lax.iota(jnp.int32, LANES) + h * HROWS)
    pltpu.sync_copy(bias_hbm, bias_v)

    @pl.when(s >= 8)
    def _():
        pltpu.sync_copy(
            zer_hbm.at[pl.ds(pl.multiple_of((s - 8) * OROWS, 8), OROWS), :],
            accum.at[pl.ds(pl.multiple_of((s - 8) * OROWS, 8), OROWS), :])

    pltpu.async_copy(tab_hbm.at[pl.ds(pl.multiple_of(s * V, 8), V)],
                     sub_v, sem_tab)

    plsc.subcore_barrier()

    def do_feature(f, first):
        if first:
            pltpu.make_async_copy(
                tab_hbm.at[pl.ds(pl.multiple_of(f * V, 8), V)],
                sub_v, sem_tab).wait()
        else:
            pltpu.sync_copy(tab_hbm.at[pl.ds(pl.multiple_of(f * V, 8), V)],
                            sub_v)

        def idx_slice(h):
            return xt_hbm.at[
                pl.ds(pl.multiple_of(f * B + c * BH + h * HB, 8), HB)]

        pltpu.async_copy(idx_slice(0), idx_v.at[0], sem_idx)
        for h in range(NH):
            pltpu.make_async_copy(idx_slice(h), idx_v.at[h % 2],
                                  sem_idx).wait()
            if h + 1 < NH:
                pltpu.async_copy(idx_slice(h + 1), idx_v.at[(h + 1) % 2],
                                 sem_idx)

            def gather_row(r, _):
                for l in range(128 // LANES):
                    iv = idx_v[h % 2, pl.ds(r * 128 + l * LANES, LANES)]
                    iv = jnp.minimum(jnp.maximum(iv, 0), V - 1)
                    part_v[r, pl.ds(l * LANES, LANES)] = (
                        plsc.load_gather(sub_v, [iv]))
                return 0
            lax.fori_loop(0, HROWS, gather_row, 0)

            pltpu.sync_copy(part_v, accum.at[iota_v.at[h]], add=True)

    do_feature(s, True)

    @pl.when(s < F - NUM_SUBCORES)
    def _():
        do_feature(s + NUM_SUBCORES, False)

    plsc.subcore_barrier()

    @pl.when(s < ROWS // OROWS)
    def _():
        pltpu.sync_copy(
            accum.at[pl.ds(pl.multiple_of(s * OROWS, 8), OROWS), :], outb_v)
        bvec = bias_v[...]

        def add_bias(r, _):
            for l in range(128 // LANES):
                outb_v[r, pl.ds(l * LANES, LANES)] = (
                    outb_v[r, pl.ds(l * LANES, LANES)] + bvec)
            return 0
        lax.fori_loop(0, OROWS, add_bias, 0)

        row0 = pl.multiple_of(c * ROWS + s * OROWS, 8)
        pltpu.sync_copy(outb_v, out_hbm.at[pl.ds(row0, OROWS), :])


@jax.jit
def _run(xt, tab, bias16, zer):
    mesh = plsc.VectorSubcoreMesh(
        core_axis_name="c", subcore_axis_name="s",
        num_cores=NUM_CORES, num_subcores=NUM_SUBCORES)
    return pl.kernel(
        _lookup_body,
        out_type=jax.ShapeDtypeStruct((B // 128, 128), jnp.float32),
        mesh=mesh,
        compiler_params=pltpu.CompilerParams(needs_layout_passes=False),
        scratch_types=[
            pltpu.VMEM((V,), jnp.float32),
            pltpu.VMEM((2, HB), jnp.int32),
            pltpu.VMEM((HROWS, 128), jnp.float32),
            pltpu.VMEM((NH, LANES), jnp.int32),
            pltpu.VMEM((LANES,), jnp.float32),
            pltpu.VMEM((OROWS, 128), jnp.float32),
            pltpu.VMEM_SHARED((ROWS, 128), jnp.float32),
            pltpu.SemaphoreType.DMA,
            pltpu.SemaphoreType.DMA,
        ],
    )(xt, tab, bias16, zer)


def kernel(x, weights_embed, bias):
    xt = x.T.reshape(-1)
    tab = weights_embed.reshape(-1)
    bias16 = jnp.broadcast_to(bias, (LANES,))
    zer = jnp.zeros((ROWS, 128), jnp.float32)
    out = _run(xt, tab, bias16, zer)
    return out.reshape(B, 1)


# speedup vs baseline: 1.0221x; 1.0221x over previous
import jax
import jax.numpy as jnp
from jax import lax
from jax.experimental import pallas as pl
from jax.experimental.pallas import tpu as pltpu
from jax.experimental.pallas import tpu_sc as plsc

NUM_CORES = 2
NUM_SUBCORES = 16
LANES = 16

B = 16384
F = 26
V = 100000
BH = B // NUM_CORES
ROWS = BH // 128
OROWS = ROWS // 8
NH = 4
HROWS = ROWS // NH
HB = BH // NH


def _lookup_body(xt_hbm, tab_hbm, bias_hbm, zer_hbm, out_hbm,
                 sub_v, idx_v, part_v, iota_v, bias_v, outb_v,
                 accum, sem_tab):
    c = lax.axis_index("c")
    s = lax.axis_index("s")

    for h in range(NH):
        iota_v[h, pl.ds(0, LANES)] = (
            lax.iota(jnp.int32, LANES) + h * HROWS)
    pltpu.sync_copy(bias_hbm, bias_v)

    @pl.when(s >= 8)
    def _():
        pltpu.sync_copy(
            zer_hbm.at[pl.ds(pl.multiple_of((s - 8) * OROWS, 8), OROWS), :],
            accum.at[pl.ds(pl.multiple_of((s - 8) * OROWS, 8), OROWS), :])

    pltpu.async_copy(tab_hbm.at[pl.ds(pl.multiple_of(s * V, 8), V)],
                     sub_v, sem_tab)

    plsc.subcore_barrier()

    def do_feature(f, first):
        if first:
            pltpu.make_async_copy(
                tab_hbm.at[pl.ds(pl.multiple_of(f * V, 8), V)],
                sub_v, sem_tab).wait()
        else:
            pltpu.sync_copy(tab_hbm.at[pl.ds(pl.multiple_of(f * V, 8), V)],
                            sub_v)

        for h in range(NH):
            pltpu.sync_copy(
                xt_hbm.at[pl.ds(pl.multiple_of(f * B + c * BH + h * HB, 8),
                                HB)],
                idx_v)

            def gather_row(r, _):
                for l in range(128 // LANES):
                    iv = idx_v[pl.ds(r * 128 + l * LANES, LANES)]
                    iv = jnp.minimum(jnp.maximum(iv, 0), V - 1)
                    part_v[r, pl.ds(l * LANES, LANES)] = (
                        plsc.load_gather(sub_v, [iv]))
                return 0
            lax.fori_loop(0, HROWS, gather_row, 0)

            pltpu.sync_copy(part_v, accum.at[iota_v.at[h]], add=True)

    do_feature(s, True)

    @pl.when(s < F - NUM_SUBCORES)
    def _():
        do_feature(s + NUM_SUBCORES, False)

    plsc.subcore_barrier()

    @pl.when(s < ROWS // OROWS)
    def _():
        pltpu.sync_copy(
            accum.at[pl.ds(pl.multiple_of(s * OROWS, 8), OROWS), :], outb_v)
        bvec = bias_v[...]

        def add_bias(r, _):
            for l in range(128 // LANES):
                outb_v[r, pl.ds(l * LANES, LANES)] = (
                    outb_v[r, pl.ds(l * LANES, LANES)] + bvec)
            return 0
        lax.fori_loop(0, OROWS, add_bias, 0)

        row0 = pl.multiple_of(c * ROWS + s * OROWS, 8)
        pltpu.sync_copy(outb_v, out_hbm.at[pl.ds(row0, OROWS), :])


@jax.jit
def _run(xt, tab, bias16, zer):
    mesh = plsc.VectorSubcoreMesh(
        core_axis_name="c", subcore_axis_name="s",
        num_cores=NUM_CORES, num_subcores=NUM_SUBCORES)
    return pl.kernel(
        _lookup_body,
        out_type=jax.ShapeDtypeStruct((B // 128, 128), jnp.float32),
        mesh=mesh,
        compiler_params=pltpu.CompilerParams(needs_layout_passes=False),
        scratch_types=[
            pltpu.VMEM((V,), jnp.float32),
            pltpu.VMEM((HB,), jnp.int32),
            pltpu.VMEM((HROWS, 128), jnp.float32),
            pltpu.VMEM((NH, LANES), jnp.int32),
            pltpu.VMEM((LANES,), jnp.float32),
            pltpu.VMEM((OROWS, 128), jnp.float32),
            pltpu.VMEM_SHARED((ROWS, 128), jnp.float32),
            pltpu.SemaphoreType.DMA,
        ],
    )(xt, tab, bias16, zer)


def kernel(x, weights_embed, bias):
    xt = x.T.reshape(-1)
    tab = weights_embed.reshape(-1)
    bias16 = jnp.broadcast_to(bias, (LANES,))
    zer = jnp.zeros((ROWS, 128), jnp.float32)
    out = _run(xt, tab, bias16, zer)
    return out.reshape(B, 1)


# speedup vs baseline: 1.0334x; 1.0111x over previous
import jax
import jax.numpy as jnp
from jax import lax
from jax.experimental import pallas as pl
from jax.experimental.pallas import tpu as pltpu
from jax.experimental.pallas import tpu_sc as plsc

NUM_CORES = 2
NUM_SUBCORES = 16
LANES = 16

B = 16384
F = 26
V = 100000
BH = B // NUM_CORES
ROWS = BH // 128
OROWS = ROWS // 8
NH = 4
HROWS = ROWS // NH
HB = BH // NH


def _lookup_body(xt_hbm, tab_hbm, zer_hbm, out_hbm,
                 sub_v, idx_v, part_v, iota_v,
                 accum, sem_tab):
    c = lax.axis_index("c")
    s = lax.axis_index("s")

    for h in range(NH):
        iota_v[h, pl.ds(0, LANES)] = (
            lax.iota(jnp.int32, LANES) + h * HROWS)

    @pl.when(s >= 8)
    def _():
        pltpu.sync_copy(
            zer_hbm.at[pl.ds(pl.multiple_of((s - 8) * OROWS, 8), OROWS), :],
            accum.at[pl.ds(pl.multiple_of((s - 8) * OROWS, 8), OROWS), :])

    pltpu.async_copy(tab_hbm.at[pl.ds(pl.multiple_of(s * V, 8), V)],
                     sub_v, sem_tab)

    plsc.subcore_barrier()

    def do_feature(f, first):
        if first:
            pltpu.make_async_copy(
                tab_hbm.at[pl.ds(pl.multiple_of(f * V, 8), V)],
                sub_v, sem_tab).wait()
        else:
            pltpu.sync_copy(tab_hbm.at[pl.ds(pl.multiple_of(f * V, 8), V)],
                            sub_v)

        for h in range(NH):
            pltpu.sync_copy(
                xt_hbm.at[pl.ds(pl.multiple_of(f * B + c * BH + h * HB, 8),
                                HB)],
                idx_v)

            def gather_row(r, _):
                for l in range(128 // LANES):
                    iv = idx_v[pl.ds(r * 128 + l * LANES, LANES)]
                    iv = jnp.minimum(jnp.maximum(iv, 0), V - 1)
                    part_v[r, pl.ds(l * LANES, LANES)] = (
                        plsc.load_gather(sub_v, [iv]))
                return 0
            lax.fori_loop(0, HROWS, gather_row, 0)

            pltpu.sync_copy(part_v, accum.at[iota_v.at[h]], add=True)

    do_feature(s, True)

    @pl.when(s < F - NUM_SUBCORES)
    def _():
        do_feature(s + NUM_SUBCORES, False)

    plsc.subcore_barrier()

    @pl.when(s < ROWS // OROWS)
    def _():
        row0 = pl.multiple_of(c * ROWS + s * OROWS, 8)
        pltpu.sync_copy(
            accum.at[pl.ds(pl.multiple_of(s * OROWS, 8), OROWS), :],
            out_hbm.at[pl.ds(row0, OROWS), :])


@jax.jit
def _run(xt, tab, zer):
    mesh = plsc.VectorSubcoreMesh(
        core_axis_name="c", subcore_axis_name="s",
        num_cores=NUM_CORES, num_subcores=NUM_SUBCORES)
    return pl.kernel(
        _lookup_body,
        out_type=jax.ShapeDtypeStruct((B // 128, 128), jnp.float32),
        mesh=mesh,
        compiler_params=pltpu.CompilerParams(needs_layout_passes=False),
        scratch_types=[
            pltpu.VMEM((V,), jnp.float32),
            pltpu.VMEM((HB,), jnp.int32),
            pltpu.VMEM((HROWS, 128), jnp.float32),
            pltpu.VMEM((NH, LANES), jnp.int32),
            pltpu.VMEM_SHARED((ROWS, 128), jnp.float32),
            pltpu.SemaphoreType.DMA,
        ],
    )(xt, tab, zer)


def kernel(x, weights_embed, bias):
    xt = x.T.reshape(-1)
    tab = weights_embed.reshape(-1)
    zer = jnp.broadcast_to(bias, (ROWS, 128))
    out = _run(xt, tab, zer)
    return out.reshape(B, 1)
